# baseline (speedup 1.0000x reference)
import functools

import jax
import jax.numpy as jnp
from jax import lax
from jax.experimental import pallas as pl
from jax.experimental.pallas import tpu as pltpu
from jax.experimental.pallas import tpu_sc as plsc

N = 10000
E = 320000
F = 128
H = 64
G = 32

NP = 10240
EP = 327680
K = 128
NC = 2
NS = 16
STRIPE = NP // NS
ZCH = 128
ST = 16

f32 = jnp.float32
i32 = jnp.int32


def _sc_mesh():
    return plsc.VectorSubcoreMesh(
        core_axis_name="c", subcore_axis_name="s", num_cores=NC, num_subcores=NS
    )


def _zero_acc(zeros_h, buf, acc, r0):
    pltpu.sync_copy(zeros_h, buf)
    for t in range(STRIPE // ZCH):
        pltpu.sync_copy(buf, acc.at[pl.ds(r0 + t * ZCH, ZCH)])


def _dump_acc(acc, buf, out, out_base, r0):
    for t in range(STRIPE // ZCH):
        pltpu.sync_copy(acc.at[pl.ds(r0 + t * ZCH, ZCH)], buf)
        pltpu.sync_copy(buf, out.at[pl.ds(out_base + r0 + t * ZCH, ZCH)])


def _scatter_run(hs, src2, dst2, row0, nchunks,
                 srcv, dstv, buf0, buf1, sem0, sem1, acc):
    def issue(j, buf, sem):
        pltpu.async_copy(hs.at[srcv.at[j]], buf, sem)

    def wait(j, buf, sem):
        pltpu.make_async_copy(hs.at[srcv.at[j]], buf, sem).wait()

    def stage(st, carry):
        pltpu.sync_copy(src2.at[pl.ds(row0 + st * ST, ST)], srcv)
        pltpu.sync_copy(dst2.at[pl.ds(row0 + st * ST, ST)], dstv)
        issue(0, buf0, sem0)
        issue(1, buf1, sem1)

        def pair(q, inner):
            for o, buf, sem in ((0, buf0, sem0), (1, buf1, sem1)):
                j = q * 2 + o
                wait(j, buf, sem)
                pltpu.sync_copy(buf, acc.at[dstv.at[j]], add=True)

                @pl.when(j + 2 < ST)
                def _():
                    issue(j + 2, buf, sem)

            return inner

        lax.fori_loop(0, ST // 2, pair, 0)
        return carry

    lax.fori_loop(0, nchunks // ST, stage, 0)


def _mk_deg_kernel():
    C = EP // K // (NC * NS)

    @functools.partial(
        pl.kernel,
        out_type=jax.ShapeDtypeStruct((NC * NP, 16), f32),
        mesh=_sc_mesh(),
        compiler_params=pltpu.CompilerParams(use_tc_tiling_on_sc=False),
        scratch_types=[
            pltpu.VMEM((ST, K), i32),
            pltpu.VMEM((K, 16), f32),
            pltpu.VMEM((ZCH, 16), f32),
            pltpu.VMEM_SHARED((NP, 16), f32),
        ],
    )
    def deg_kernel(dst2, ones_h, zeros_h, out, dstv, onesv, zbuf, acc):
        c = lax.axis_index("c")
        s = lax.axis_index("s")
        r0 = s * STRIPE
        row0 = (c * NS + s) * C
        pltpu.sync_copy(ones_h, onesv)
        _zero_acc(zeros_h, zbuf, acc, r0)
        plsc.subcore_barrier()

        def stage(st, carry):
            pltpu.sync_copy(dst2.at[pl.ds(row0 + st * ST, ST)], dstv)

            def step(j, inner):
                pltpu.sync_copy(onesv, acc.at[dstv.at[j]], add=True)
                return inner

            lax.fori_loop(0, ST, step, 0)
            return carry

        lax.fori_loop(0, C // ST, stage, 0)
        plsc.subcore_barrier()
        _dump_acc(acc, zbuf, out, c * NP, r0)

    return deg_kernel


def _mk_scatter_edges(hc):
    C = EP // K // (NC * NS)

    @functools.partial(
        pl.kernel,
        out_type=jax.ShapeDtypeStruct((NC * NP, hc), f32),
        mesh=_sc_mesh(),
        compiler_params=pltpu.CompilerParams(use_tc_tiling_on_sc=False),
        scratch_types=[
            pltpu.VMEM((ST, K), i32),
            pltpu.VMEM((ST, K), i32),
            pltpu.VMEM((K, hc), f32),
            pltpu.VMEM((K, hc), f32),
            pltpu.VMEM_SHARED((NP, hc), f32),
            pltpu.SemaphoreType.DMA,
            pltpu.SemaphoreType.DMA,
        ],
    )
    def scatter_kernel(hs, src2, dst2, zeros_h, out,
                       srcv, dstv, buf0, buf1, acc, sem0, sem1):
        c = lax.axis_index("c")
        s = lax.axis_index("s")
        r0 = s * STRIPE
        row0 = (c * NS + s) * C
        _zero_acc(zeros_h, buf0, acc, r0)
        plsc.subcore_barrier()
        _scatter_run(hs, src2, dst2, row0, C,
                     srcv, dstv, buf0, buf1, sem0, sem1, acc)
        plsc.subcore_barrier()
        _dump_acc(acc, buf0, out, c * NP, r0)

    return scatter_kernel


def _mk_scatter_cols():
    C = EP // K // NS

    @functools.partial(
        pl.kernel,
        out_type=jax.ShapeDtypeStruct((NC * NP, 128), f32),
        mesh=_sc_mesh(),
        compiler_params=pltpu.CompilerParams(use_tc_tiling_on_sc=False),
        scratch_types=[
            pltpu.VMEM((ST, K), i32),
            pltpu.VMEM((ST, K), i32),
            pltpu.VMEM((K, 128), f32),
            pltpu.VMEM((K, 128), f32),
            pltpu.VMEM_SHARED((NP, 128), f32),
            pltpu.SemaphoreType.DMA,
            pltpu.SemaphoreType.DMA,
        ],
    )
    def scatter3_kernel(hsa, hsb, src2, dst2, zeros_h, out,
                        srcv, dstv, buf0, buf1, acc, sem0, sem1):
        c = lax.axis_index("c")
        s = lax.axis_index("s")
        r0 = s * STRIPE
        row0 = s * C
        _zero_acc(zeros_h, buf0, acc, r0)
        plsc.subcore_barrier()

        @pl.when(c == 0)
        def _():
            _scatter_run(hsa, src2, dst2, row0, C,
                         srcv, dstv, buf0, buf1, sem0, sem1, acc)

        @pl.when(c == 1)
        def _():
            _scatter_run(hsb, src2, dst2, row0, C,
                         srcv, dstv, buf0, buf1, sem0, sem1, acc)

        plsc.subcore_barrier()
        _dump_acc(acc, buf0, out, c * NP, r0)

    return scatter3_kernel


_BLK = 1024
_NBLK = NP // _BLK


def _dinv_of(degp_ref):
    deg = degp_ref[0, :, 0:1] + degp_ref[1, :, 0:1] + 1.0
    return lax.rsqrt(deg)


def _lrelu(t):
    return jnp.where(t > 0, t, 0.01 * t)


def _tc_first(x_ref, w_ref, degp_ref, out_ref):
    dinv = _dinv_of(degp_ref)
    h = jnp.dot(x_ref[...], w_ref[...], preferred_element_type=f32)
    out_ref[...] = h * dinv


def _tc_mid(s_ref, hs_ref, degp_ref, b_ref, w_ref, out_ref):
    dinv = _dinv_of(degp_ref)
    s = s_ref[0] + s_ref[1]
    xn = _lrelu(dinv * (s + hs_ref[...]) + b_ref[...])
    out_ref[...] = jnp.dot(xn, w_ref[...], preferred_element_type=f32) * dinv


def _tc_mid2(s_ref, hs_ref, degp_ref, b_ref, w_ref, outa_ref, outb_ref):
    dinv = _dinv_of(degp_ref)
    s = s_ref[0] + s_ref[1]
    xn = _lrelu(dinv * (s + hs_ref[...]) + b_ref[...])
    hs3 = jnp.dot(xn, w_ref[...], preferred_element_type=f32) * dinv
    outa_ref[...] = hs3[:, :128]
    outb_ref[...] = hs3[:, 128:]


def _tc_final(s3_ref, hsa_ref, hsb_ref, degp_ref, b3_ref, batch_ref,
              wc1_ref, bc1_ref, wc2_ref, bc2_ref, wc3_ref, bc3_ref,
              out_ref, acc_ref):
    i = pl.program_id(0)

    @pl.when(i == 0)
    def _():
        acc_ref[...] = jnp.full((G, 4 * H), -1e30, f32)

    dinv = _dinv_of(degp_ref)
    fa = dinv * (s3_ref[0] + hsa_ref[...])
    fb = dinv * (s3_ref[1] + hsb_ref[...])
    f_blk = jnp.concatenate((fa, fb), axis=1) + b3_ref[...]
    batch = batch_ref[0]
    for g in range(G):
        vals = jnp.where(batch == g, f_blk, -1e30)
        m = jnp.max(vals, axis=0, keepdims=True)
        acc_ref[pl.ds(g, 1), :] = jnp.maximum(acc_ref[pl.ds(g, 1), :], m)

    @pl.when(i == _NBLK - 1)
    def _():
        f1 = acc_ref[...]
        fcat = jnp.concatenate((f1, f1), axis=1)
        z = jnp.maximum(
            jnp.dot(fcat, wc1_ref[...], preferred_element_type=f32)
            + bc1_ref[...], 0.0)
        z = jnp.maximum(
            jnp.dot(z, wc2_ref[...], preferred_element_type=f32)
            + bc2_ref[...], 0.0)
        out_ref[...] = (
            jnp.dot(z, wc3_ref[...], preferred_element_type=f32) + bc3_ref[...])


def _full(shape):
    nd = len(shape)
    return pl.BlockSpec(shape, lambda i, _nd=nd: (0,) * _nd)


def _rows(width):
    return pl.BlockSpec((_BLK, width), lambda i: (i, 0))


_DEGP_SPEC = pl.BlockSpec((2, _BLK, 16), lambda i: (0, i, 0))


def kernel(x, edge_index, batch, W1, b1, W2, b2, W3, b3,
           Wc1, bc1, Wc2, bc2, Wc3, bc3):
    pad_e = EP - E
    src2 = jnp.concatenate(
        [edge_index[0], jnp.zeros((pad_e,), i32)]).reshape(EP // K, K)
    dst2 = jnp.concatenate(
        [edge_index[1], N + jnp.arange(pad_e, dtype=i32) % (NP - N)]
    ).reshape(EP // K, K)
    xp = jnp.concatenate([x, jnp.zeros((NP - N, F), f32)])
    batch3 = jnp.concatenate(
        [batch, jnp.full((NP - N,), G, i32)]).reshape(_NBLK, _BLK, 1)
    ones16 = jnp.ones((K, 16), f32)
    z16 = jnp.zeros((ZCH, 16), f32)
    z64 = jnp.zeros((ZCH, H), f32)
    z128 = jnp.zeros((ZCH, 128), f32)

    degp = _mk_deg_kernel()(dst2, ones16, z16).reshape(NC, NP, 16)

    hs1 = pl.pallas_call(
        _tc_first,
        grid=(_NBLK,),
        in_specs=[_rows(F), _full((F, H)), _DEGP_SPEC],
        out_specs=_rows(H),
        out_shape=jax.ShapeDtypeStruct((NP, H), f32),
    )(xp, W1, degp)

    S1 = _mk_scatter_edges(H)(hs1, src2, dst2, z64).reshape(NC, NP, H)

    hs2 = pl.pallas_call(
        _tc_mid,
        grid=(_NBLK,),
        in_specs=[
            pl.BlockSpec((2, _BLK, H), lambda i: (0, i, 0)),
            _rows(H), _DEGP_SPEC, _full((1, H)), _full((H, 2 * H)),
        ],
        out_specs=_rows(2 * H),
        out_shape=jax.ShapeDtypeStruct((NP, 2 * H), f32),
    )(S1, hs1, degp, b1.reshape(1, H), W2)

    S2 = _mk_scatter_edges(2 * H)(hs2, src2, dst2, z128).reshape(NC, NP, 2 * H)

    hs3a, hs3b = pl.pallas_call(
        _tc_mid2,
        grid=(_NBLK,),
        in_specs=[
            pl.BlockSpec((2, _BLK, 2 * H), lambda i: (0, i, 0)),
            _rows(2 * H), _DEGP_SPEC, _full((1, 2 * H)), _full((2 * H, 4 * H)),
        ],
        out_specs=[_rows(2 * H), _rows(2 * H)],
        out_shape=[
            jax.ShapeDtypeStruct((NP, 2 * H), f32),
            jax.ShapeDtypeStruct((NP, 2 * H), f32),
        ],
    )(S2, hs2, degp, b2.reshape(1, 2 * H), W3)

    S3 = _mk_scatter_cols()(hs3a, hs3b, src2, dst2, z128).reshape(NC, NP, 128)

    out = pl.pallas_call(
        _tc_final,
        grid=(_NBLK,),
        in_specs=[
            pl.BlockSpec((2, _BLK, 128), lambda i: (0, i, 0)),
            _rows(128), _rows(128), _DEGP_SPEC, _full((1, 4 * H)),
            pl.BlockSpec((1, _BLK, 1), lambda i: (i, 0, 0)),
            _full((8 * H, 1024)), _full((1, 1024)),
            _full((1024, 512)), _full((1, 512)),
            _full((512, 4)), _full((1, 4)),
        ],
        out_specs=pl.BlockSpec((G, 4), lambda i: (0, 0)),
        out_shape=jax.ShapeDtypeStruct((G, 4), f32),
        scratch_shapes=[pltpu.VMEM((G, 4 * H), f32)],
    )(S3, hs3a, hs3b, degp, b3.reshape(1, 4 * H), batch3,
      Wc1, bc1.reshape(1, 1024), Wc2, bc2.reshape(1, 512),
      Wc3, bc3.reshape(1, 4))

    return out


# speedup vs baseline: 2.4338x; 2.4338x over previous
import functools

import jax
import jax.numpy as jnp
from jax import lax
from jax.experimental import pallas as pl
from jax.experimental.pallas import tpu as pltpu
from jax.experimental.pallas import tpu_sc as plsc

N = 10000
E = 320000
F = 128
H = 64
G = 32

NP = 10240
EP = 327680
K = 128
NC = 2
NS = 16
STRIPE = NP // NS
ZCH = 128
ST = 16

f32 = jnp.float32
i32 = jnp.int32


def _sc_mesh():
    return plsc.VectorSubcoreMesh(
        core_axis_name="c", subcore_axis_name="s", num_cores=NC, num_subcores=NS
    )


def _zero_acc(zeros_h, buf, acc, r0):
    pltpu.sync_copy(zeros_h, buf)
    for t in range(STRIPE // ZCH):
        pltpu.sync_copy(buf, acc.at[pl.ds(r0 + t * ZCH, ZCH)])


def _dump_acc(acc, buf, out, out_base, r0):
    for t in range(STRIPE // ZCH):
        pltpu.sync_copy(acc.at[pl.ds(r0 + t * ZCH, ZCH)], buf)
        pltpu.sync_copy(buf, out.at[pl.ds(out_base + r0 + t * ZCH, ZCH)])


def _scatter_run(hs, src2, dst2, row0, nchunks,
                 srcv, dstv, buf0, buf1, sem0, sem1, acc):
    def issue(j, buf, sem):
        pltpu.async_copy(hs.at[srcv.at[j]], buf, sem)

    def wait(j, buf, sem):
        pltpu.make_async_copy(hs.at[srcv.at[j]], buf, sem).wait()

    def stage(st, carry):
        pltpu.sync_copy(src2.at[pl.ds(row0 + st * ST, ST)], srcv)
        pltpu.sync_copy(dst2.at[pl.ds(row0 + st * ST, ST)], dstv)
        issue(0, buf0, sem0)
        issue(1, buf1, sem1)

        def pair(q, inner):
            for o, buf, sem in ((0, buf0, sem0), (1, buf1, sem1)):
                j = q * 2 + o
                wait(j, buf, sem)
                pltpu.sync_copy(buf, acc.at[dstv.at[j]], add=True)

                @pl.when(j + 2 < ST)
                def _():
                    issue(j + 2, buf, sem)

            return inner

        lax.fori_loop(0, ST // 2, pair, 0)
        return carry

    lax.fori_loop(0, nchunks // ST, stage, 0)


def _mk_deg_kernel():
    C = EP // K // (NC * NS)

    @functools.partial(
        pl.kernel,
        out_type=jax.ShapeDtypeStruct((NC * NP, 16), f32),
        mesh=_sc_mesh(),
        compiler_params=pltpu.CompilerParams(use_tc_tiling_on_sc=False),
        scratch_types=[
            pltpu.VMEM((ST, K), i32),
            pltpu.VMEM((K, 16), f32),
            pltpu.VMEM((ZCH, 16), f32),
            pltpu.VMEM_SHARED((NP, 16), f32),
        ],
    )
    def deg_kernel(dst2, ones_h, zeros_h, out, dstv, onesv, zbuf, acc):
        c = lax.axis_index("c")
        s = lax.axis_index("s")
        r0 = s * STRIPE
        row0 = (c * NS + s) * C
        pltpu.sync_copy(ones_h, onesv)
        _zero_acc(zeros_h, zbuf, acc, r0)
        plsc.subcore_barrier()

        def stage(st, carry):
            pltpu.sync_copy(dst2.at[pl.ds(row0 + st * ST, ST)], dstv)

            def step(j, inner):
                pltpu.sync_copy(onesv, acc.at[dstv.at[j]], add=True)
                return inner

            lax.fori_loop(0, ST, step, 0)
            return carry

        lax.fori_loop(0, C // ST, stage, 0)
        plsc.subcore_barrier()
        _dump_acc(acc, zbuf, out, c * NP, r0)

    return deg_kernel


def _mk_scatter_edges(hc):
    C = EP // K // (NC * NS)

    @functools.partial(
        pl.kernel,
        out_type=jax.ShapeDtypeStruct((NC * NP, hc), f32),
        mesh=_sc_mesh(),
        compiler_params=pltpu.CompilerParams(use_tc_tiling_on_sc=False),
        scratch_types=[
            pltpu.VMEM((ST, K), i32),
            pltpu.VMEM((ST, K), i32),
            pltpu.VMEM((K, hc), f32),
            pltpu.VMEM((K, hc), f32),
            pltpu.VMEM_SHARED((NP, hc), f32),
            pltpu.SemaphoreType.DMA,
            pltpu.SemaphoreType.DMA,
        ],
    )
    def scatter_kernel(hs, src2, dst2, zeros_h, out,
                       srcv, dstv, buf0, buf1, acc, sem0, sem1):
        c = lax.axis_index("c")
        s = lax.axis_index("s")
        r0 = s * STRIPE
        row0 = (c * NS + s) * C
        _zero_acc(zeros_h, buf0, acc, r0)
        plsc.subcore_barrier()
        _scatter_run(hs, src2, dst2, row0, C,
                     srcv, dstv, buf0, buf1, sem0, sem1, acc)
        plsc.subcore_barrier()
        _dump_acc(acc, buf0, out, c * NP, r0)

    return scatter_kernel


def _mk_scatter_cols():
    C = EP // K // NS

    @functools.partial(
        pl.kernel,
        out_type=jax.ShapeDtypeStruct((NC * NP, 128), f32),
        mesh=_sc_mesh(),
        compiler_params=pltpu.CompilerParams(use_tc_tiling_on_sc=False),
        scratch_types=[
            pltpu.VMEM((ST, K), i32),
            pltpu.VMEM((ST, K), i32),
            pltpu.VMEM((K, 128), f32),
            pltpu.VMEM((K, 128), f32),
            pltpu.VMEM_SHARED((NP, 128), f32),
            pltpu.SemaphoreType.DMA,
            pltpu.SemaphoreType.DMA,
        ],
    )
    def scatter3_kernel(hsa, hsb, src2, dst2, zeros_h, out,
                        srcv, dstv, buf0, buf1, acc, sem0, sem1):
        c = lax.axis_index("c")
        s = lax.axis_index("s")
        r0 = s * STRIPE
        row0 = s * C
        _zero_acc(zeros_h, buf0, acc, r0)
        plsc.subcore_barrier()

        @pl.when(c == 0)
        def _():
            _scatter_run(hsa, src2, dst2, row0, C,
                         srcv, dstv, buf0, buf1, sem0, sem1, acc)

        @pl.when(c == 1)
        def _():
            _scatter_run(hsb, src2, dst2, row0, C,
                         srcv, dstv, buf0, buf1, sem0, sem1, acc)

        plsc.subcore_barrier()
        _dump_acc(acc, buf0, out, c * NP, r0)

    return scatter3_kernel


_BLK = 1024
_NBLK = NP // _BLK


def _dinv_of(degp_ref):
    deg = degp_ref[0, :, 0:1] + degp_ref[1, :, 0:1] + 1.0
    return lax.rsqrt(deg)


def _lrelu(t):
    return jnp.where(t > 0, t, 0.01 * t)


def _tc_first(x_ref, w_ref, degp_ref, out_ref):
    dinv = _dinv_of(degp_ref)
    h = jnp.dot(x_ref[...], w_ref[...], preferred_element_type=f32)
    out_ref[...] = h * dinv


def _valid_rows():
    rowid = lax.broadcasted_iota(i32, (_BLK, 1), 0) + pl.program_id(0) * _BLK
    return rowid < N


def _tc_mid(s_ref, hs_ref, degp_ref, b_ref, w_ref, out_ref):
    dinv = _dinv_of(degp_ref)
    s = s_ref[0] + s_ref[1]
    xn = _lrelu(dinv * (s + hs_ref[...]) + b_ref[...])
    hs_next = jnp.dot(xn, w_ref[...], preferred_element_type=f32) * dinv
    out_ref[...] = jnp.where(_valid_rows(), hs_next, 0.0)


def _tc_mid2(s_ref, hs_ref, degp_ref, b_ref, w_ref, outa_ref, outb_ref):
    dinv = _dinv_of(degp_ref)
    s = s_ref[0] + s_ref[1]
    xn = _lrelu(dinv * (s + hs_ref[...]) + b_ref[...])
    hs3 = jnp.dot(xn, w_ref[...], preferred_element_type=f32) * dinv
    hs3 = jnp.where(_valid_rows(), hs3, 0.0)
    outa_ref[...] = hs3[:, :128]
    outb_ref[...] = hs3[:, 128:]


def _tc_final(s3_ref, hsa_ref, hsb_ref, degp_ref, b3_ref, batch_ref,
              wc1_ref, bc1_ref, wc2_ref, bc2_ref, wc3_ref, bc3_ref,
              out_ref, acc_ref):
    i = pl.program_id(0)

    @pl.when(i == 0)
    def _():
        acc_ref[...] = jnp.full((G, 4 * H), -1e30, f32)

    dinv = _dinv_of(degp_ref)
    fa = dinv * (s3_ref[0] + hsa_ref[...])
    fb = dinv * (s3_ref[1] + hsb_ref[...])
    f_blk = jnp.concatenate((fa, fb), axis=1) + b3_ref[...]
    batch = batch_ref[0]
    for g in range(G):
        vals = jnp.where(batch == g, f_blk, -1e30)
        m = jnp.max(vals, axis=0, keepdims=True)
        acc_ref[pl.ds(g, 1), :] = jnp.maximum(acc_ref[pl.ds(g, 1), :], m)

    @pl.when(i == _NBLK - 1)
    def _():
        f1 = acc_ref[...]
        fcat = jnp.concatenate((f1, f1), axis=1)
        z = jnp.maximum(
            jnp.dot(fcat, wc1_ref[...], preferred_element_type=f32)
            + bc1_ref[...], 0.0)
        z = jnp.maximum(
            jnp.dot(z, wc2_ref[...], preferred_element_type=f32)
            + bc2_ref[...], 0.0)
        out_ref[...] = (
            jnp.dot(z, wc3_ref[...], preferred_element_type=f32) + bc3_ref[...])


def _full(shape):
    nd = len(shape)
    return pl.BlockSpec(shape, lambda i, _nd=nd: (0,) * _nd)


def _rows(width):
    return pl.BlockSpec((_BLK, width), lambda i: (i, 0))


_DEGP_SPEC = pl.BlockSpec((2, _BLK, 16), lambda i: (0, i, 0))


def kernel(x, edge_index, batch, W1, b1, W2, b2, W3, b3,
           Wc1, bc1, Wc2, bc2, Wc3, bc3):
    pad_e = EP - E
    pad_ar = jnp.arange(pad_e, dtype=i32)
    src2 = jnp.concatenate(
        [edge_index[0], N + pad_ar % (NP - N)]).reshape(EP // K, K)
    dst2 = jnp.concatenate(
        [edge_index[1], (pad_ar * 131) % NP]).reshape(EP // K, K)
    dst2_deg = jnp.concatenate(
        [edge_index[1], N + pad_ar % (NP - N)]).reshape(EP // K, K)
    xp = jnp.concatenate([x, jnp.zeros((NP - N, F), f32)])
    batch3 = jnp.concatenate(
        [batch, jnp.full((NP - N,), G, i32)]).reshape(_NBLK, _BLK, 1)
    ones16 = jnp.ones((K, 16), f32)
    z16 = jnp.zeros((ZCH, 16), f32)
    z64 = jnp.zeros((ZCH, H), f32)
    z128 = jnp.zeros((ZCH, 128), f32)

    degp = _mk_deg_kernel()(dst2_deg, ones16, z16).reshape(NC, NP, 16)

    hs1 = pl.pallas_call(
        _tc_first,
        grid=(_NBLK,),
        in_specs=[_rows(F), _full((F, H)), _DEGP_SPEC],
        out_specs=_rows(H),
        out_shape=jax.ShapeDtypeStruct((NP, H), f32),
    )(xp, W1, degp)

    S1 = _mk_scatter_edges(H)(hs1, src2, dst2, z64).reshape(NC, NP, H)

    hs2 = pl.pallas_call(
        _tc_mid,
        grid=(_NBLK,),
        in_specs=[
            pl.BlockSpec((2, _BLK, H), lambda i: (0, i, 0)),
            _rows(H), _DEGP_SPEC, _full((1, H)), _full((H, 2 * H)),
        ],
        out_specs=_rows(2 * H),
        out_shape=jax.ShapeDtypeStruct((NP, 2 * H), f32),
    )(S1, hs1, degp, b1.reshape(1, H), W2)

    S2 = _mk_scatter_edges(2 * H)(hs2, src2, dst2, z128).reshape(NC, NP, 2 * H)

    hs3a, hs3b = pl.pallas_call(
        _tc_mid2,
        grid=(_NBLK,),
        in_specs=[
            pl.BlockSpec((2, _BLK, 2 * H), lambda i: (0, i, 0)),
            _rows(2 * H), _DEGP_SPEC, _full((1, 2 * H)), _full((2 * H, 4 * H)),
        ],
        out_specs=[_rows(2 * H), _rows(2 * H)],
        out_shape=[
            jax.ShapeDtypeStruct((NP, 2 * H), f32),
            jax.ShapeDtypeStruct((NP, 2 * H), f32),
        ],
    )(S2, hs2, degp, b2.reshape(1, 2 * H), W3)

    S3 = _mk_scatter_cols()(hs3a, hs3b, src2, dst2, z128).reshape(NC, NP, 128)

    out = pl.pallas_call(
        _tc_final,
        grid=(_NBLK,),
        in_specs=[
            pl.BlockSpec((2, _BLK, 128), lambda i: (0, i, 0)),
            _rows(128), _rows(128), _DEGP_SPEC, _full((1, 4 * H)),
            pl.BlockSpec((1, _BLK, 1), lambda i: (i, 0, 0)),
            _full((8 * H, 1024)), _full((1, 1024)),
            _full((1024, 512)), _full((1, 512)),
            _full((512, 4)), _full((1, 4)),
        ],
        out_specs=pl.BlockSpec((G, 4), lambda i: (0, 0)),
        out_shape=jax.ShapeDtypeStruct((G, 4), f32),
        scratch_shapes=[pltpu.VMEM((G, 4 * H), f32)],
    )(S3, hs3a, hs3b, degp, b3.reshape(1, 4 * H), batch3,
      Wc1, bc1.reshape(1, 1024), Wc2, bc2.reshape(1, 512),
      Wc3, bc3.reshape(1, 4))

    return out


# speedup vs baseline: 3.1304x; 1.2862x over previous
import functools

import jax
import jax.numpy as jnp
from jax import lax
from jax.experimental import pallas as pl
from jax.experimental.pallas import tpu as pltpu
from jax.experimental.pallas import tpu_sc as plsc

N = 10000
E = 320000
F = 128
H = 64
G = 32

NP = 10240
EP = 327680
K = 128
NC = 2
NS = 16
STRIPE = NP // NS
ZCH = 128
ST = 16

f32 = jnp.float32
i32 = jnp.int32


def _sc_mesh():
    return plsc.VectorSubcoreMesh(
        core_axis_name="c", subcore_axis_name="s", num_cores=NC, num_subcores=NS
    )


def _zero_acc(zeros_h, buf, acc, r0):
    pltpu.sync_copy(zeros_h, buf)
    for t in range(STRIPE // ZCH):
        pltpu.sync_copy(buf, acc.at[pl.ds(r0 + t * ZCH, ZCH)])


def _dump_acc(acc, buf, out, out_base, r0):
    for t in range(STRIPE // ZCH):
        pltpu.sync_copy(acc.at[pl.ds(r0 + t * ZCH, ZCH)], buf)
        pltpu.sync_copy(buf, out.at[pl.ds(out_base + r0 + t * ZCH, ZCH)])


def _scatter_run(hs, src2, dst2, row0, nchunks,
                 srcv, dstv, buf0, buf1, sem0, sem1, acc):
    def issue(j, buf, sem):
        pltpu.async_copy(hs.at[srcv.at[j]], buf, sem)

    def wait(j, buf, sem):
        pltpu.make_async_copy(hs.at[srcv.at[j]], buf, sem).wait()

    def stage(st, carry):
        pltpu.sync_copy(src2.at[pl.ds(row0 + st * ST, ST)], srcv)
        pltpu.sync_copy(dst2.at[pl.ds(row0 + st * ST, ST)], dstv)
        issue(0, buf0, sem0)
        issue(1, buf1, sem1)

        def pair(q, inner):
            for o, buf, sem in ((0, buf0, sem0), (1, buf1, sem1)):
                j = q * 2 + o
                wait(j, buf, sem)
                pltpu.sync_copy(buf, acc.at[dstv.at[j]], add=True)

                @pl.when(j + 2 < ST)
                def _():
                    issue(j + 2, buf, sem)

            return inner

        lax.fori_loop(0, ST // 2, pair, 0)
        return carry

    lax.fori_loop(0, nchunks // ST, stage, 0)


def _mk_deg_kernel():
    C = EP // K // (NC * NS)

    @functools.partial(
        pl.kernel,
        out_type=jax.ShapeDtypeStruct((NC * NP, 16), f32),
        mesh=_sc_mesh(),
        compiler_params=pltpu.CompilerParams(use_tc_tiling_on_sc=False),
        scratch_types=[
            pltpu.VMEM((ST, K), i32),
            pltpu.VMEM((K, 16), f32),
            pltpu.VMEM((ZCH, 16), f32),
            pltpu.VMEM_SHARED((NP, 16), f32),
        ],
    )
    def deg_kernel(dst2, ones_h, zeros_h, out, dstv, onesv, zbuf, acc):
        c = lax.axis_index("c")
        s = lax.axis_index("s")
        r0 = s * STRIPE
        row0 = (c * NS + s) * C
        pltpu.sync_copy(ones_h, onesv)
        _zero_acc(zeros_h, zbuf, acc, r0)
        plsc.subcore_barrier()

        def stage(st, carry):
            pltpu.sync_copy(dst2.at[pl.ds(row0 + st * ST, ST)], dstv)

            def step(j, inner):
                pltpu.sync_copy(onesv, acc.at[dstv.at[j]], add=True)
                return inner

            lax.fori_loop(0, ST, step, 0)
            return carry

        lax.fori_loop(0, C // ST, stage, 0)
        plsc.subcore_barrier()
        _dump_acc(acc, zbuf, out, c * NP, r0)

    return deg_kernel


def _mk_scatter_edges(hc):
    C = EP // K // (NC * NS)

    @functools.partial(
        pl.kernel,
        out_type=jax.ShapeDtypeStruct((NC * NP, hc), f32),
        mesh=_sc_mesh(),
        compiler_params=pltpu.CompilerParams(use_tc_tiling_on_sc=False),
        scratch_types=[
            pltpu.VMEM((ST, K), i32),
            pltpu.VMEM((ST, K), i32),
            pltpu.VMEM((K, hc), f32),
            pltpu.VMEM((K, hc), f32),
            pltpu.VMEM_SHARED((NP, hc), f32),
            pltpu.SemaphoreType.DMA,
            pltpu.SemaphoreType.DMA,
        ],
    )
    def scatter_kernel(hs, src2, dst2, zeros_h, out,
                       srcv, dstv, buf0, buf1, acc, sem0, sem1):
        c = lax.axis_index("c")
        s = lax.axis_index("s")
        r0 = s * STRIPE
        row0 = (c * NS + s) * C
        _zero_acc(zeros_h, buf0, acc, r0)
        plsc.subcore_barrier()
        _scatter_run(hs, src2, dst2, row0, C,
                     srcv, dstv, buf0, buf1, sem0, sem1, acc)
        plsc.subcore_barrier()
        _dump_acc(acc, buf0, out, c * NP, r0)

    return scatter_kernel


_BLK = 1024
_NBLK = NP // _BLK


def _dinv_of(degp_ref):
    deg = degp_ref[0, :, 0:1] + degp_ref[1, :, 0:1] + 1.0
    return lax.rsqrt(deg)


def _lrelu(t):
    return jnp.where(t > 0, t, 0.01 * t)


def _valid_rows():
    rowid = lax.broadcasted_iota(i32, (_BLK, 1), 0) + pl.program_id(0) * _BLK
    return rowid < N


def _tc_first(x_ref, w_ref, degp_ref, out_ref):
    dinv = _dinv_of(degp_ref)
    h = jnp.dot(x_ref[...], w_ref[...], preferred_element_type=f32)
    out_ref[...] = h * dinv


def _tc_a(s_ref, hs_ref, degp_ref, b_ref, out_ref):
    dinv = _dinv_of(degp_ref)
    s = s_ref[0] + s_ref[1]
    xn = _lrelu(dinv * (s + hs_ref[...]) + b_ref[...])
    out_ref[...] = jnp.where(_valid_rows(), xn * dinv, 0.0)


def _tc_b(t_ref, xs_ref, degp_ref, b_ref, w_ref, out_ref):
    dinv = _dinv_of(degp_ref)
    t = t_ref[0] + t_ref[1] + xs_ref[...]
    agg = dinv * jnp.dot(t, w_ref[...], preferred_element_type=f32) + b_ref[...]
    out_ref[...] = jnp.where(_valid_rows(), _lrelu(agg) * dinv, 0.0)


def _tc_final(t3_ref, xs_ref, degp_ref, b3_ref, w3_ref, batch_ref,
              wc1_ref, bc1_ref, wc2_ref, bc2_ref, wc3_ref, bc3_ref,
              out_ref, acc_ref):
    i = pl.program_id(0)

    @pl.when(i == 0)
    def _():
        acc_ref[...] = jnp.full((G, 4 * H), -1e30, f32)

    dinv = _dinv_of(degp_ref)
    t = t3_ref[0] + t3_ref[1] + xs_ref[...]
    f_blk = (dinv * jnp.dot(t, w3_ref[...], preferred_element_type=f32)
             + b3_ref[...])
    batch = batch_ref[0]
    for g in range(G):
        vals = jnp.where(batch == g, f_blk, -1e30)
        m = jnp.max(vals, axis=0, keepdims=True)
        acc_ref[pl.ds(g, 1), :] = jnp.maximum(acc_ref[pl.ds(g, 1), :], m)

    @pl.when(i == _NBLK - 1)
    def _():
        f1 = acc_ref[...]
        fcat = jnp.concatenate((f1, f1), axis=1)
        z = jnp.maximum(
            jnp.dot(fcat, wc1_ref[...], preferred_element_type=f32)
            + bc1_ref[...], 0.0)
        z = jnp.maximum(
            jnp.dot(z, wc2_ref[...], preferred_element_type=f32)
            + bc2_ref[...], 0.0)
        out_ref[...] = (
            jnp.dot(z, wc3_ref[...], preferred_element_type=f32) + bc3_ref[...])


def _full(shape):
    nd = len(shape)
    return pl.BlockSpec(shape, lambda i, _nd=nd: (0,) * _nd)


def _rows(width):
    return pl.BlockSpec((_BLK, width), lambda i: (i, 0))


_DEGP_SPEC = pl.BlockSpec((2, _BLK, 16), lambda i: (0, i, 0))


def kernel(x, edge_index, batch, W1, b1, W2, b2, W3, b3,
           Wc1, bc1, Wc2, bc2, Wc3, bc3):
    pad_e = EP - E
    pad_ar = jnp.arange(pad_e, dtype=i32)
    src2 = jnp.concatenate(
        [edge_index[0], N + pad_ar % (NP - N)]).reshape(EP // K, K)
    dst2 = jnp.concatenate(
        [edge_index[1], (pad_ar * 131) % NP]).reshape(EP // K, K)
    dst2_deg = jnp.concatenate(
        [edge_index[1], N + pad_ar % (NP - N)]).reshape(EP // K, K)
    xp = jnp.concatenate([x, jnp.zeros((NP - N, F), f32)])
    batch3 = jnp.concatenate(
        [batch, jnp.full((NP - N,), G, i32)]).reshape(_NBLK, _BLK, 1)
    ones16 = jnp.ones((K, 16), f32)
    z16 = jnp.zeros((ZCH, 16), f32)
    z64 = jnp.zeros((ZCH, H), f32)
    z128 = jnp.zeros((ZCH, 128), f32)

    degp = _mk_deg_kernel()(dst2_deg, ones16, z16).reshape(NC, NP, 16)

    hs1 = pl.pallas_call(
        _tc_first,
        grid=(_NBLK,),
        in_specs=[_rows(F), _full((F, H)), _DEGP_SPEC],
        out_specs=_rows(H),
        out_shape=jax.ShapeDtypeStruct((NP, H), f32),
    )(xp, W1, degp)

    S1 = _mk_scatter_edges(H)(hs1, src2, dst2, z64).reshape(NC, NP, H)

    x2s = pl.pallas_call(
        _tc_a,
        grid=(_NBLK,),
        in_specs=[
            pl.BlockSpec((2, _BLK, H), lambda i: (0, i, 0)),
            _rows(H), _DEGP_SPEC, _full((1, H)),
        ],
        out_specs=_rows(H),
        out_shape=jax.ShapeDtypeStruct((NP, H), f32),
    )(S1, hs1, degp, b1.reshape(1, H))

    T2 = _mk_scatter_edges(H)(x2s, src2, dst2, z64).reshape(NC, NP, H)

    x3s = pl.pallas_call(
        _tc_b,
        grid=(_NBLK,),
        in_specs=[
            pl.BlockSpec((2, _BLK, H), lambda i: (0, i, 0)),
            _rows(H), _DEGP_SPEC, _full((1, 2 * H)), _full((H, 2 * H)),
        ],
        out_specs=_rows(2 * H),
        out_shape=jax.ShapeDtypeStruct((NP, 2 * H), f32),
    )(T2, x2s, degp, b2.reshape(1, 2 * H), W2)

    T3 = _mk_scatter_edges(2 * H)(x3s, src2, dst2, z128).reshape(NC, NP, 2 * H)

    out = pl.pallas_call(
        _tc_final,
        grid=(_NBLK,),
        in_specs=[
            pl.BlockSpec((2, _BLK, 2 * H), lambda i: (0, i, 0)),
            _rows(2 * H), _DEGP_SPEC, _full((1, 4 * H)), _full((2 * H, 4 * H)),
            pl.BlockSpec((1, _BLK, 1), lambda i: (i, 0, 0)),
            _full((8 * H, 1024)), _full((1, 1024)),
            _full((1024, 512)), _full((1, 512)),
            _full((512, 4)), _full((1, 4)),
        ],
        out_specs=pl.BlockSpec((G, 4), lambda i: (0, 0)),
        out_shape=jax.ShapeDtypeStruct((G, 4), f32),
        scratch_shapes=[pltpu.VMEM((G, 4 * H), f32)],
    )(T3, x3s, degp, b3.reshape(1, 4 * H), W3, batch3,
      Wc1, bc1.reshape(1, 1024), Wc2, bc2.reshape(1, 512),
      Wc3, bc3.reshape(1, 4))

    return out


# speedup vs baseline: 3.3465x; 1.0690x over previous
import functools

import jax
import jax.numpy as jnp
from jax import lax
from jax.experimental import pallas as pl
from jax.experimental.pallas import tpu as pltpu
from jax.experimental.pallas import tpu_sc as plsc

N = 10000
E = 320000
F = 128
H = 64
G = 32

NP = 10240
EP = 327680
K = 128
NC = 2
NS = 16
STRIPE = NP // NS
ZCH = 128
ST = 16

f32 = jnp.float32
i32 = jnp.int32


def _sc_mesh():
    return plsc.VectorSubcoreMesh(
        core_axis_name="c", subcore_axis_name="s", num_cores=NC, num_subcores=NS
    )


def _zero_acc(zeros_h, buf, acc, r0):
    pltpu.sync_copy(zeros_h, buf)
    for t in range(STRIPE // ZCH):
        pltpu.sync_copy(buf, acc.at[pl.ds(r0 + t * ZCH, ZCH)])


def _dump_acc(acc, buf, out, out_base, r0):
    for t in range(STRIPE // ZCH):
        pltpu.sync_copy(acc.at[pl.ds(r0 + t * ZCH, ZCH)], buf)
        pltpu.sync_copy(buf, out.at[pl.ds(out_base + r0 + t * ZCH, ZCH)])


def _scatter_run(hs, src2, dst2, row0, nchunks,
                 srcv, dstv, buf0, buf1, sem0, sem1, acc):
    def issue(j, buf, sem):
        pltpu.async_copy(hs.at[srcv.at[j]], buf, sem)

    def wait(j, buf, sem):
        pltpu.make_async_copy(hs.at[srcv.at[j]], buf, sem).wait()

    def stage(st, carry):
        pltpu.sync_copy(src2.at[pl.ds(row0 + st * ST, ST)], srcv)
        pltpu.sync_copy(dst2.at[pl.ds(row0 + st * ST, ST)], dstv)
        issue(0, buf0, sem0)
        issue(1, buf1, sem1)

        def pair(q, inner):
            for o, buf, sem in ((0, buf0, sem0), (1, buf1, sem1)):
                j = q * 2 + o
                wait(j, buf, sem)
                pltpu.sync_copy(buf, acc.at[dstv.at[j]], add=True)

                @pl.when(j + 2 < ST)
                def _():
                    issue(j + 2, buf, sem)

            return inner

        lax.fori_loop(0, ST // 2, pair, 0)
        return carry

    lax.fori_loop(0, nchunks // ST, stage, 0)


def _mk_deg_kernel():
    C = EP // K // (NC * NS)

    @functools.partial(
        pl.kernel,
        out_type=jax.ShapeDtypeStruct((NC * NP, 16), f32),
        mesh=_sc_mesh(),
        compiler_params=pltpu.CompilerParams(use_tc_tiling_on_sc=False),
        scratch_types=[
            pltpu.VMEM((ST, K), i32),
            pltpu.VMEM((K, 16), f32),
            pltpu.VMEM((ZCH, 16), f32),
            pltpu.VMEM_SHARED((NP, 16), f32),
        ],
    )
    def deg_kernel(dst2, ones_h, zeros_h, out, dstv, onesv, zbuf, acc):
        c = lax.axis_index("c")
        s = lax.axis_index("s")
        r0 = s * STRIPE
        row0 = (c * NS + s) * C
        pltpu.sync_copy(ones_h, onesv)
        _zero_acc(zeros_h, zbuf, acc, r0)
        plsc.subcore_barrier()

        def stage(st, carry):
            pltpu.sync_copy(dst2.at[pl.ds(row0 + st * ST, ST)], dstv)

            def step(j, inner):
                pltpu.sync_copy(onesv, acc.at[dstv.at[j]], add=True)
                return inner

            lax.fori_loop(0, ST, step, 0)
            return carry

        lax.fori_loop(0, C // ST, stage, 0)
        plsc.subcore_barrier()
        _dump_acc(acc, zbuf, out, c * NP, r0)

    return deg_kernel


def _mk_scatter_edges(hc):
    C = EP // K // (NC * NS)

    @functools.partial(
        pl.kernel,
        out_type=jax.ShapeDtypeStruct((NC * NP, hc), f32),
        mesh=_sc_mesh(),
        compiler_params=pltpu.CompilerParams(use_tc_tiling_on_sc=False),
        scratch_types=[
            pltpu.VMEM((ST, K), i32),
            pltpu.VMEM((ST, K), i32),
            pltpu.VMEM((K, hc), f32),
            pltpu.VMEM((K, hc), f32),
            pltpu.VMEM_SHARED((NP, hc), f32),
            pltpu.SemaphoreType.DMA,
            pltpu.SemaphoreType.DMA,
        ],
    )
    def scatter_kernel(hs, src2, dst2, zeros_h, out,
                       srcv, dstv, buf0, buf1, acc, sem0, sem1):
        c = lax.axis_index("c")
        s = lax.axis_index("s")
        r0 = s * STRIPE
        row0 = (c * NS + s) * C
        _zero_acc(zeros_h, buf0, acc, r0)
        plsc.subcore_barrier()
        _scatter_run(hs, src2, dst2, row0, C,
                     srcv, dstv, buf0, buf1, sem0, sem1, acc)
        plsc.subcore_barrier()
        _dump_acc(acc, buf0, out, c * NP, r0)

    return scatter_kernel


_BLK = 1024
_NBLK = NP // _BLK


def _dinv_of(d0_ref, d1_ref):
    deg = d0_ref[:, 0:1] + d1_ref[:, 0:1] + 1.0
    return lax.rsqrt(deg)


def _lrelu(t):
    return jnp.where(t > 0, t, 0.01 * t)


def _valid_rows():
    rowid = lax.broadcasted_iota(i32, (_BLK, 1), 0) + pl.program_id(0) * _BLK
    return rowid < N


def _tc_first(x_ref, w_ref, d0_ref, d1_ref, out_ref):
    dinv = _dinv_of(d0_ref, d1_ref)
    h = jnp.dot(x_ref[...], w_ref[...], preferred_element_type=f32)
    out_ref[...] = h * dinv


def _tc_a(s0_ref, s1_ref, hs_ref, d0_ref, d1_ref, b_ref, out_ref):
    dinv = _dinv_of(d0_ref, d1_ref)
    s = s0_ref[...] + s1_ref[...]
    xn = _lrelu(dinv * (s + hs_ref[...]) + b_ref[...])
    out_ref[...] = jnp.where(_valid_rows(), xn * dinv, 0.0)


def _tc_b(t0_ref, t1_ref, xs_ref, d0_ref, d1_ref, b_ref, w_ref, out_ref):
    dinv = _dinv_of(d0_ref, d1_ref)
    t = t0_ref[...] + t1_ref[...] + xs_ref[...]
    agg = dinv * jnp.dot(t, w_ref[...], preferred_element_type=f32) + b_ref[...]
    out_ref[...] = jnp.where(_valid_rows(), _lrelu(agg) * dinv, 0.0)


def _tc_final(t30_ref, t31_ref, xs_ref, d0_ref, d1_ref, b3_ref, w3_ref,
              batch_ref, wc1_ref, bc1_ref, wc2_ref, bc2_ref, wc3_ref, bc3_ref,
              out_ref, acc_ref):
    i = pl.program_id(0)

    @pl.when(i == 0)
    def _():
        acc_ref[...] = jnp.full((G, 4 * H), -1e30, f32)

    dinv = _dinv_of(d0_ref, d1_ref)
    t = t30_ref[...] + t31_ref[...] + xs_ref[...]
    f_blk = (dinv * jnp.dot(t, w3_ref[...], preferred_element_type=f32)
             + b3_ref[...])
    batch = batch_ref[0]
    bmin = jnp.min(batch)
    bmax = jnp.max(batch)
    for g in range(G):
        @pl.when((g >= bmin) & (g <= bmax))
        def _(g=g):
            vals = jnp.where(batch == g, f_blk, -1e30)
            m = jnp.max(vals, axis=0, keepdims=True)
            acc_ref[pl.ds(g, 1), :] = jnp.maximum(acc_ref[pl.ds(g, 1), :], m)

    @pl.when(i == _NBLK - 1)
    def _():
        f1 = acc_ref[...]
        fcat = jnp.concatenate((f1, f1), axis=1)
        z = jnp.maximum(
            jnp.dot(fcat, wc1_ref[...], preferred_element_type=f32)
            + bc1_ref[...], 0.0)
        z = jnp.maximum(
            jnp.dot(z, wc2_ref[...], preferred_element_type=f32)
            + bc2_ref[...], 0.0)
        out_ref[...] = (
            jnp.dot(z, wc3_ref[...], preferred_element_type=f32) + bc3_ref[...])


def _full(shape):
    nd = len(shape)
    return pl.BlockSpec(shape, lambda i, _nd=nd: (0,) * _nd)


def _rows(width):
    return pl.BlockSpec((_BLK, width), lambda i: (i, 0))


def _rows_hi(width):
    return pl.BlockSpec((_BLK, width), lambda i: (i + _NBLK, 0))


def kernel(x, edge_index, batch, W1, b1, W2, b2, W3, b3,
           Wc1, bc1, Wc2, bc2, Wc3, bc3):
    pad_e = EP - E
    pad_ar = jnp.arange(pad_e, dtype=i32)
    src2 = jnp.concatenate(
        [edge_index[0], N + pad_ar % (NP - N)]).reshape(EP // K, K)
    dst2 = jnp.concatenate(
        [edge_index[1], (pad_ar * 131) % NP]).reshape(EP // K, K)
    dst2_deg = jnp.concatenate(
        [edge_index[1], N + pad_ar % (NP - N)]).reshape(EP // K, K)
    xp = jnp.concatenate([x, jnp.zeros((NP - N, F), f32)])
    batch3 = jnp.concatenate(
        [batch, jnp.full((NP - N,), G, i32)]).reshape(_NBLK, _BLK, 1)
    ones16 = jnp.ones((K, 16), f32)
    z16 = jnp.zeros((ZCH, 16), f32)
    z64 = jnp.zeros((ZCH, H), f32)
    z128 = jnp.zeros((ZCH, 128), f32)

    degp = _mk_deg_kernel()(dst2_deg, ones16, z16)

    hs1 = pl.pallas_call(
        _tc_first,
        grid=(_NBLK,),
        in_specs=[_rows(F), _full((F, H)), _rows(16), _rows_hi(16)],
        out_specs=_rows(H),
        out_shape=jax.ShapeDtypeStruct((NP, H), f32),
    )(xp, W1, degp, degp)

    S1 = _mk_scatter_edges(H)(hs1, src2, dst2, z64)

    x2s = pl.pallas_call(
        _tc_a,
        grid=(_NBLK,),
        in_specs=[
            _rows(H), _rows_hi(H),
            _rows(H), _rows(16), _rows_hi(16), _full((1, H)),
        ],
        out_specs=_rows(H),
        out_shape=jax.ShapeDtypeStruct((NP, H), f32),
    )(S1, S1, hs1, degp, degp, b1.reshape(1, H))

    T2 = _mk_scatter_edges(H)(x2s, src2, dst2, z64)

    x3s = pl.pallas_call(
        _tc_b,
        grid=(_NBLK,),
        in_specs=[
            _rows(H), _rows_hi(H),
            _rows(H), _rows(16), _rows_hi(16), _full((1, 2 * H)),
            _full((H, 2 * H)),
        ],
        out_specs=_rows(2 * H),
        out_shape=jax.ShapeDtypeStruct((NP, 2 * H), f32),
    )(T2, T2, x2s, degp, degp, b2.reshape(1, 2 * H), W2)

    T3 = _mk_scatter_edges(2 * H)(x3s, src2, dst2, z128)

    out = pl.pallas_call(
        _tc_final,
        grid=(_NBLK,),
        in_specs=[
            _rows(2 * H), _rows_hi(2 * H),
            _rows(2 * H), _rows(16), _rows_hi(16), _full((1, 4 * H)),
            _full((2 * H, 4 * H)),
            pl.BlockSpec((1, _BLK, 1), lambda i: (i, 0, 0)),
            _full((8 * H, 1024)), _full((1, 1024)),
            _full((1024, 512)), _full((1, 512)),
            _full((512, 4)), _full((1, 4)),
        ],
        out_specs=pl.BlockSpec((G, 4), lambda i: (0, 0)),
        out_shape=jax.ShapeDtypeStruct((G, 4), f32),
        scratch_shapes=[pltpu.VMEM((G, 4 * H), f32)],
    )(T3, T3, x3s, degp, degp, b3.reshape(1, 4 * H), W3, batch3,
      Wc1, bc1.reshape(1, 1024), Wc2, bc2.reshape(1, 512),
      Wc3, bc3.reshape(1, 4))

    return out
